# baseline (speedup 1.0000x reference)
import functools

import jax
import jax.numpy as jnp
from jax import lax
from jax.experimental import pallas as pl
from jax.experimental.pallas import tpu as pltpu
from jax.experimental.pallas import tpu_sc as plsc

S, T = 4096, 200
D = 64
V = 1000000
NC, NS = 2, 16
NW = NC * NS
SPW = S // NW
BLK = 8
NBLK = SPW // BLK
NBUF = 2

_vector_mesh = plsc.VectorSubcoreMesh(
    core_axis_name="core", subcore_axis_name="subcore"
)


CA = 160
NCH = V // CA


@jax.jit
def _widen_sc(table):
    @functools.partial(
        pl.kernel,
        out_type=jax.ShapeDtypeStruct((V, 2 * D), jnp.float32),
        mesh=_vector_mesh,
        scratch_types=[
            pltpu.VMEM((2, CA, D), jnp.float32),
            pltpu.VMEM((CA, 2 * D), jnp.float32),
            pltpu.SemaphoreType.DMA((2,)),
            pltpu.SemaphoreType.DMA,
        ],
    )
    def kern(tab_hbm, wide_hbm, a_v, b_v, insem, outsem):
        wid = lax.axis_index("subcore") * NC + lax.axis_index("core")

        pltpu.async_copy(
            tab_hbm.at[pl.ds(wid * CA, CA)], a_v.at[0], insem.at[0]
        )

        @pl.loop(0, 196, step=2)
        def _(c):
            for q in range(2):
                cc = c + q
                ch = cc * NW + wid

                @pl.when(ch < NCH)
                def _():
                    off = ch * CA
                    pltpu.make_async_copy(
                        tab_hbm.at[pl.ds(off, CA)], a_v.at[q], insem.at[q]
                    ).wait()

                    @pl.when(ch + NW < NCH)
                    def _():
                        pltpu.async_copy(
                            tab_hbm.at[pl.ds(off + NW * CA, CA)],
                            a_v.at[1 - q],
                            insem.at[1 - q],
                        )

                    @pl.when(cc > 0)
                    def _():
                        pltpu.make_async_copy(
                            b_v, wide_hbm.at[pl.ds(off, CA)], outsem
                        ).wait()

                    @pl.loop(0, CA // 8)
                    def _(r):
                        for l in range(8):
                            for k in range(4):
                                b_v[r * 8 + l, pl.ds(k * 16, 16)] = a_v[
                                    q, r * 8 + l, pl.ds(k * 16, 16)
                                ]

                    pltpu.async_copy(
                        b_v, wide_hbm.at[pl.ds(off, CA)], outsem
                    )

        pltpu.make_async_copy(
            b_v, wide_hbm.at[pl.ds(0, CA)], outsem
        ).wait()

    return kern(table)


@jax.jit
def _gather_sc(wide, tok):
    @functools.partial(
        pl.kernel,
        out_type=jax.ShapeDtypeStruct((S, T, D), jnp.float32),
        mesh=_vector_mesh,
        scratch_types=[
            pltpu.VMEM((NBUF, BLK, T), jnp.int32),
            pltpu.VMEM((NBUF, 1, T, 2 * D), jnp.float32),
            pltpu.VMEM((NBUF, 1, T, D), jnp.float32),
            pltpu.SemaphoreType.DMA((NBUF,)),
            pltpu.SemaphoreType.DMA((NBUF,)),
            pltpu.SemaphoreType.DMA((NBUF,)),
        ],
    )
    def kern(wide_hbm, tok_hbm, out_hbm, idx_v, rows_v, sel_v, isem, gsem,
             osem):
        wid = lax.axis_index("subcore") * NC + lax.axis_index("core")
        base = wid * SPW

        for b in range(NBUF):
            pltpu.async_copy(
                tok_hbm.at[pl.ds(base + b * BLK, BLK)], idx_v.at[b],
                isem.at[b],
            )

        @pl.loop(0, NBLK, step=NBUF)
        def _(i):
            for b in range(NBUF):
                s0 = base + (i + b) * BLK

                pltpu.make_async_copy(
                    tok_hbm.at[pl.ds(s0, BLK)], idx_v.at[b], isem.at[b]
                ).wait()

                for g0, gn in ((0, 128), (128, T - 128)):
                    pltpu.async_copy(
                        wide_hbm.at[idx_v.at[b, 0, pl.ds(g0, gn)]],
                        rows_v.at[0, 0, pl.ds(g0, gn)],
                        gsem.at[0],
                    )

                for j in range(BLK):
                    p = j % NBUF

                    for g0, gn in ((0, 128), (128, T - 128)):
                        pltpu.make_async_copy(
                            wide_hbm.at[idx_v.at[b, j, pl.ds(g0, gn)]],
                            rows_v.at[p, 0, pl.ds(g0, gn)],
                            gsem.at[p],
                        ).wait()

                    if j + 1 < BLK:
                        for g0, gn in ((0, 128), (128, T - 128)):
                            pltpu.async_copy(
                                wide_hbm.at[
                                    idx_v.at[b, j + 1, pl.ds(g0, gn)]
                                ],
                                rows_v.at[1 - p, 0, pl.ds(g0, gn)],
                                gsem.at[1 - p],
                            )

                    if b == 0 and j < NBUF:
                        @pl.when(i > 0)
                        def _():
                            pltpu.make_async_copy(
                                sel_v.at[p],
                                out_hbm.at[pl.ds(s0, 1)],
                                osem.at[p],
                            ).wait()
                    else:
                        pltpu.make_async_copy(
                            sel_v.at[p],
                            out_hbm.at[pl.ds(s0, 1)],
                            osem.at[p],
                        ).wait()

                    @pl.loop(0, T // 8)
                    def _(tt):
                        for l in range(8):
                            for k in range(4):
                                sel_v[p, 0, tt * 8 + l, pl.ds(k * 16, 16)] = (
                                    rows_v[
                                        p, 0, tt * 8 + l, pl.ds(k * 16, 16)
                                    ]
                                )

                    pltpu.async_copy(
                        sel_v.at[p],
                        out_hbm.at[pl.ds(s0 + j, 1)],
                        osem.at[p],
                    )

                @pl.when(i + NBUF < NBLK)
                def _():
                    pltpu.async_copy(
                        tok_hbm.at[pl.ds(s0 + NBUF * BLK, BLK)],
                        idx_v.at[b],
                        isem.at[b],
                    )

        for p in range(NBUF):
            pltpu.make_async_copy(
                sel_v.at[p], out_hbm.at[pl.ds(base, 1)], osem.at[p]
            ).wait()

    return kern(wide, tok)


def kernel(tokenized_sentence, table):
    wide = _widen_sc(table)
    return _gather_sc(wide, tokenized_sentence)


# speedup vs baseline: 1.1821x; 1.1821x over previous
import functools

import jax
import jax.numpy as jnp
from jax import lax
from jax.experimental import pallas as pl
from jax.experimental.pallas import tpu as pltpu
from jax.experimental.pallas import tpu_sc as plsc

S, T = 4096, 200
D = 64
V = 1000000
NC, NS = 2, 16
NW = NC * NS
SPW = S // NW
BLK = 8
NBLK = SPW // BLK
NBUF = 2

_vector_mesh = plsc.VectorSubcoreMesh(
    core_axis_name="core", subcore_axis_name="subcore"
)


@jax.jit
def _gather_sc(wide, tok):
    @functools.partial(
        pl.kernel,
        out_type=jax.ShapeDtypeStruct((S, T, D), jnp.float32),
        mesh=_vector_mesh,
        scratch_types=[
            pltpu.VMEM((NBUF, BLK, T), jnp.int32),
            pltpu.VMEM((NBUF, 1, T, 2 * D), jnp.float32),
            pltpu.VMEM((NBUF, 1, T, D), jnp.float32),
            pltpu.SemaphoreType.DMA((NBUF,)),
            pltpu.SemaphoreType.DMA((NBUF,)),
            pltpu.SemaphoreType.DMA((NBUF,)),
        ],
    )
    def kern(wide_hbm, tok_hbm, out_hbm, idx_v, rows_v, sel_v, isem, gsem,
             osem):
        wid = lax.axis_index("subcore") * NC + lax.axis_index("core")
        base = wid * SPW

        for b in range(NBUF):
            pltpu.async_copy(
                tok_hbm.at[pl.ds(base + b * BLK, BLK)], idx_v.at[b],
                isem.at[b],
            )

        @pl.loop(0, NBLK, step=NBUF)
        def _(i):
            for b in range(NBUF):
                s0 = base + (i + b) * BLK

                pltpu.make_async_copy(
                    tok_hbm.at[pl.ds(s0, BLK)], idx_v.at[b], isem.at[b]
                ).wait()

                for g0, gn in ((0, 128), (128, T - 128)):
                    pltpu.async_copy(
                        wide_hbm.at[idx_v.at[b, 0, pl.ds(g0, gn)]],
                        rows_v.at[0, 0, pl.ds(g0, gn)],
                        gsem.at[0],
                    )

                for j in range(BLK):
                    p = j % NBUF

                    for g0, gn in ((0, 128), (128, T - 128)):
                        pltpu.make_async_copy(
                            wide_hbm.at[idx_v.at[b, j, pl.ds(g0, gn)]],
                            rows_v.at[p, 0, pl.ds(g0, gn)],
                            gsem.at[p],
                        ).wait()

                    if j + 1 < BLK:
                        for g0, gn in ((0, 128), (128, T - 128)):
                            pltpu.async_copy(
                                wide_hbm.at[
                                    idx_v.at[b, j + 1, pl.ds(g0, gn)]
                                ],
                                rows_v.at[1 - p, 0, pl.ds(g0, gn)],
                                gsem.at[1 - p],
                            )

                    if b == 0 and j < NBUF:
                        @pl.when(i > 0)
                        def _():
                            pltpu.make_async_copy(
                                sel_v.at[p],
                                out_hbm.at[pl.ds(s0, 1)],
                                osem.at[p],
                            ).wait()
                    else:
                        pltpu.make_async_copy(
                            sel_v.at[p],
                            out_hbm.at[pl.ds(s0, 1)],
                            osem.at[p],
                        ).wait()

                    @pl.loop(0, T // 8)
                    def _(tt):
                        for l in range(8):
                            for k in range(4):
                                sel_v[p, 0, tt * 8 + l, pl.ds(k * 16, 16)] = (
                                    rows_v[
                                        p, 0, tt * 8 + l, pl.ds(k * 16, 16)
                                    ]
                                )

                    pltpu.async_copy(
                        sel_v.at[p],
                        out_hbm.at[pl.ds(s0 + j, 1)],
                        osem.at[p],
                    )

                @pl.when(i + NBUF < NBLK)
                def _():
                    pltpu.async_copy(
                        tok_hbm.at[pl.ds(s0 + NBUF * BLK, BLK)],
                        idx_v.at[b],
                        isem.at[b],
                    )

        for p in range(NBUF):
            pltpu.make_async_copy(
                sel_v.at[p], out_hbm.at[pl.ds(base, 1)], osem.at[p]
            ).wait()

    return kern(wide, tok)


def kernel(tokenized_sentence, table):
    wide = jnp.pad(table, ((0, 0), (0, D)))
    return _gather_sc(wide, tokenized_sentence)
